# baseline (speedup 1.0000x reference)
import functools

import jax
import jax.numpy as jnp
from jax import lax
from jax.experimental import pallas as pl
from jax.experimental.pallas import tpu as pltpu
from jax.experimental.pallas import tpu_sc as plsc

N_NODES = 10000
D_FEAT = 128
N_EDGES = 320000
HID = 8


_TAB_BLK = 2048
N_PAD = 5 * _TAB_BLK


def _tab_body(x_ref, w_ref, b_ref, o_ref):
    tab = 2.0 * (
        jnp.dot(x_ref[...], w_ref[...], preferred_element_type=jnp.float32)
        + b_ref[...]
    )
    h16 = lax.bitcast_convert_type(
        tab.astype(jnp.bfloat16), jnp.uint16
    ).reshape(_TAB_BLK, HID, 2)
    lo = h16[:, :, 0].astype(jnp.uint32)
    hi = h16[:, :, 1].astype(jnp.uint32)
    packed = lax.bitcast_convert_type(lo | (hi << 16), jnp.int32)
    o_ref[...] = packed.T


def _make_table(x, w1cat, brow):
    return pl.pallas_call(
        _tab_body,
        grid=(N_PAD // _TAB_BLK,),
        in_specs=[
            pl.BlockSpec((_TAB_BLK, D_FEAT), lambda i: (i, 0)),
            pl.BlockSpec((D_FEAT, 2 * HID), lambda i: (0, 0)),
            pl.BlockSpec((1, 2 * HID), lambda i: (0, 0)),
        ],
        out_specs=pl.BlockSpec((HID, _TAB_BLK), lambda i: (0, i)),
        out_shape=jax.ShapeDtypeStruct((HID, N_PAD), jnp.int32),
    )(x, w1cat, brow)


def _sc_edge_mlp(tabp, src, dst, wpack, *, per_w):
    groups = per_w // 16
    mesh = plsc.VectorSubcoreMesh(core_axis_name="c", subcore_axis_name="s")

    @functools.partial(
        pl.kernel,
        mesh=mesh,
        compiler_params=pltpu.CompilerParams(
            needs_layout_passes=False, use_tc_tiling_on_sc=False),
        out_type=jax.ShapeDtypeStruct((N_EDGES,), jnp.float32),
        scratch_types=[
            pltpu.VMEM((HID, N_PAD), jnp.int32),
            pltpu.VMEM((per_w,), jnp.int32),
            pltpu.VMEM((per_w,), jnp.int32),
            pltpu.VMEM((per_w,), jnp.float32),
            pltpu.VMEM((160, 16), jnp.float32),
            pltpu.SemaphoreType.DMA,
            pltpu.SemaphoreType.DMA,
            pltpu.SemaphoreType.DMA,
            pltpu.SemaphoreType.DMA,
        ],
    )
    def sc_k(tab_h, src_h, dst_h, wpack_h, out_h,
             tabv, idx_s, idx_d, outb, wv, sem0, sem1, sem2, sem3):
        wid = lax.axis_index("s") * 2 + lax.axis_index("c")
        base = wid * per_w
        cps = [
            pltpu.async_copy(tab_h, tabv, sem0),
            pltpu.async_copy(src_h.at[pl.ds(base, per_w)], idx_s, sem1),
            pltpu.async_copy(dst_h.at[pl.ds(base, per_w)], idx_d, sem2),
            pltpu.async_copy(wpack_h, wv, sem3),
        ]
        for cp in cps:
            cp.wait()

        col = [jnp.full((16,), i, jnp.int32) for i in range(HID)]

        def edge_group_t(g):
            sv = idx_s[pl.ds(g * 16, 16)]
            dv = idx_d[pl.ds(g * 16, 16)]
            t = []
            for i in range(HID // 2):
                gs = plsc.load_gather(tabv, [col[i], sv])
                gd = plsc.load_gather(tabv, [col[HID // 2 + i], dv])
                s0, s1 = plsc.unpack(
                    plsc.bitcast(gs, jnp.bfloat16),
                    format=plsc.PackFormat.INTERLEAVED)
                d0, d1 = plsc.unpack(
                    plsc.bitcast(gd, jnp.bfloat16),
                    format=plsc.PackFormat.INTERLEAVED)
                e0 = jnp.exp(s0 + d0)
                t.append(1.0 - 2.0 / (e0 + 1.0))
                e1 = jnp.exp(s1 + d1)
                t.append(1.0 - 2.0 / (e1 + 1.0))
            return t

        def mlp_tail(ts):
            h = ts
            for wbase, bbase in ((0, 136), (64, 144)):
                nxt = [[] for _ in h]
                for j in range(HID):
                    bj = wv[bbase + j]
                    accs = [bj for _ in h]
                    for i in range(HID):
                        wij = wv[wbase + i * HID + j]
                        accs = [a + hg[i] * wij for a, hg in zip(accs, h)]
                    for k, a in enumerate(accs):
                        e = jnp.exp(a)
                        nxt[k].append(1.0 - 2.0 / (e + 1.0))
                h = nxt
            b4 = wv[152]
            outs = [b4 for _ in h]
            for i in range(HID):
                w4i = wv[128 + i]
                outs = [o + hg[i] * w4i for o, hg in zip(outs, h)]
            return outs

        pairs = groups // 2

        @plsc.parallel_loop(0, pairs, unroll=2)
        def pair_body(p):
            g0 = p * 2
            o0, o1 = mlp_tail([edge_group_t(g0), edge_group_t(g0 + 1)])
            outb[pl.ds(g0 * 16, 16)] = o0
            outb[pl.ds(g0 * 16 + 16, 16)] = o1

        if groups % 2:
            g = groups - 1
            (o_tail,) = mlp_tail([edge_group_t(g)])
            outb[pl.ds(g * 16, 16)] = o_tail

        pltpu.sync_copy(outb, out_h.at[pl.ds(base, per_w)])

    return sc_k(tabp, src, dst, wpack)


def kernel(inputs, edge_index, W1, b1, W2, b2, W3, b3, W4, b4):
    w1cat = jnp.concatenate([W1[:D_FEAT], W1[D_FEAT:]], axis=1)
    brow = jnp.concatenate([b1, jnp.zeros((HID,), jnp.float32)])[None, :]
    tabp = _make_table(inputs, w1cat, brow)

    wflat = jnp.concatenate([
        2.0 * W2.reshape(-1), 2.0 * W3.reshape(-1), W4.reshape(-1),
        2.0 * b2, 2.0 * b3, b4, jnp.zeros((7,), jnp.float32),
    ])
    wpack = jnp.broadcast_to(wflat[:, None], (160, 16))

    per_w = N_EDGES // 32
    out = _sc_edge_mlp(tabp, edge_index[0], edge_index[1], wpack,
                       per_w=per_w)
    return out.reshape(N_EDGES, 1)


# speedup vs baseline: 1.1171x; 1.1171x over previous
import functools

import jax
import jax.numpy as jnp
from jax import lax
from jax.experimental import pallas as pl
from jax.experimental.pallas import tpu as pltpu
from jax.experimental.pallas import tpu_sc as plsc

N_NODES = 10000
D_FEAT = 128
N_EDGES = 320000
HID = 8


def _tab_body(x_ref, w_ref, b_ref, o_ref):
    o_ref[...] = 2.0 * (
        jnp.dot(x_ref[...], w_ref[...], preferred_element_type=jnp.float32)
        + b_ref[...]
    )


def _make_table(x, w1cat, brow):
    return pl.pallas_call(
        _tab_body,
        out_shape=jax.ShapeDtypeStruct((N_NODES, 2 * HID), jnp.float32),
    )(x, w1cat, brow)


def _sc_edge_mlp(tabp, src, dst, wpack, *, per_w):
    groups = per_w // 16
    mesh = plsc.VectorSubcoreMesh(core_axis_name="c", subcore_axis_name="s")

    @functools.partial(
        pl.kernel,
        mesh=mesh,
        compiler_params=pltpu.CompilerParams(
            needs_layout_passes=False, use_tc_tiling_on_sc=False),
        out_type=jax.ShapeDtypeStruct((N_EDGES,), jnp.float32),
        scratch_types=[
            pltpu.VMEM((HID, N_NODES), jnp.int32),
            pltpu.VMEM((per_w,), jnp.int32),
            pltpu.VMEM((per_w,), jnp.int32),
            pltpu.VMEM((per_w,), jnp.float32),
            pltpu.VMEM((160, 16), jnp.float32),
            pltpu.SemaphoreType.DMA,
            pltpu.SemaphoreType.DMA,
            pltpu.SemaphoreType.DMA,
            pltpu.SemaphoreType.DMA,
        ],
    )
    def sc_k(tab_h, src_h, dst_h, wpack_h, out_h,
             tabv, idx_s, idx_d, outb, wv, sem0, sem1, sem2, sem3):
        wid = lax.axis_index("s") * 2 + lax.axis_index("c")
        base = wid * per_w
        cps = [
            pltpu.async_copy(tab_h, tabv, sem0),
            pltpu.async_copy(src_h.at[pl.ds(base, per_w)], idx_s, sem1),
            pltpu.async_copy(dst_h.at[pl.ds(base, per_w)], idx_d, sem2),
            pltpu.async_copy(wpack_h, wv, sem3),
        ]
        for cp in cps:
            cp.wait()

        col = [jnp.full((16,), i, jnp.int32) for i in range(HID)]

        def edge_group_t(g):
            sv = idx_s[pl.ds(g * 16, 16)]
            dv = idx_d[pl.ds(g * 16, 16)]
            t = []
            for i in range(HID // 2):
                gs = plsc.load_gather(tabv, [col[i], sv])
                gd = plsc.load_gather(tabv, [col[HID // 2 + i], dv])
                s0, s1 = plsc.unpack(
                    plsc.bitcast(gs, jnp.bfloat16),
                    format=plsc.PackFormat.INTERLEAVED)
                d0, d1 = plsc.unpack(
                    plsc.bitcast(gd, jnp.bfloat16),
                    format=plsc.PackFormat.INTERLEAVED)
                e0 = jnp.exp(s0 + d0)
                t.append(1.0 - 2.0 / (e0 + 1.0))
                e1 = jnp.exp(s1 + d1)
                t.append(1.0 - 2.0 / (e1 + 1.0))
            return t

        def mlp_tail(ts):
            h = ts
            for wbase, bbase in ((0, 136), (64, 144)):
                nxt = [[] for _ in h]
                for j in range(HID):
                    bj = wv[bbase + j]
                    accs = [bj for _ in h]
                    for i in range(HID):
                        wij = wv[wbase + i * HID + j]
                        accs = [a + hg[i] * wij for a, hg in zip(accs, h)]
                    for k, a in enumerate(accs):
                        e = jnp.exp(a)
                        nxt[k].append(1.0 - 2.0 / (e + 1.0))
                h = nxt
            b4 = wv[152]
            outs = [b4 for _ in h]
            for i in range(HID):
                w4i = wv[128 + i]
                outs = [o + hg[i] * w4i for o, hg in zip(outs, h)]
            return outs

        pairs = groups // 2

        @plsc.parallel_loop(0, pairs, unroll=2)
        def pair_body(p):
            g0 = p * 2
            o0, o1 = mlp_tail([edge_group_t(g0), edge_group_t(g0 + 1)])
            outb[pl.ds(g0 * 16, 16)] = o0
            outb[pl.ds(g0 * 16 + 16, 16)] = o1

        if groups % 2:
            g = groups - 1
            (o_tail,) = mlp_tail([edge_group_t(g)])
            outb[pl.ds(g * 16, 16)] = o_tail

        pltpu.sync_copy(outb, out_h.at[pl.ds(base, per_w)])

    return sc_k(tabp, src, dst, wpack)


def kernel(inputs, edge_index, W1, b1, W2, b2, W3, b3, W4, b4):
    w1cat = jnp.concatenate([W1[:D_FEAT], W1[D_FEAT:]], axis=1)
    brow = jnp.concatenate([b1, jnp.zeros((HID,), jnp.float32)])[None, :]
    tab = _make_table(inputs, w1cat, brow)

    h16 = lax.bitcast_convert_type(
        tab.astype(jnp.bfloat16), jnp.uint16).astype(jnp.uint32)
    packed = h16[:, 0::2] | (h16[:, 1::2] << 16)
    tabp = lax.bitcast_convert_type(packed.T, jnp.int32)

    wflat = jnp.concatenate([
        2.0 * W2.reshape(-1), 2.0 * W3.reshape(-1), W4.reshape(-1),
        2.0 * b2, 2.0 * b3, b4, jnp.zeros((7,), jnp.float32),
    ])
    wpack = jnp.broadcast_to(wflat[:, None], (160, 16))

    per_w = N_EDGES // 32
    out = _sc_edge_mlp(tabp, edge_index[0], edge_index[1], wpack,
                       per_w=per_w)
    return out.reshape(N_EDGES, 1)


# speedup vs baseline: 1.1320x; 1.0133x over previous
import functools

import jax
import jax.numpy as jnp
from jax import lax
from jax.experimental import pallas as pl
from jax.experimental.pallas import tpu as pltpu
from jax.experimental.pallas import tpu_sc as plsc

N_NODES = 10000
D_FEAT = 128
N_EDGES = 320000
HID = 8


def _tab_body(x_ref, w_ref, b_ref, o_ref):
    o_ref[...] = 2.0 * (
        jnp.dot(x_ref[...], w_ref[...], preferred_element_type=jnp.float32)
        + b_ref[...]
    )


def _make_table(x, w1cat, brow):
    return pl.pallas_call(
        _tab_body,
        out_shape=jax.ShapeDtypeStruct((N_NODES, 2 * HID), jnp.float32),
    )(x, w1cat, brow)


def _sc_edge_mlp(tabp, src, dst, wpack, *, per_w):
    groups = per_w // 16
    mesh = plsc.VectorSubcoreMesh(core_axis_name="c", subcore_axis_name="s")

    @functools.partial(
        pl.kernel,
        mesh=mesh,
        compiler_params=pltpu.CompilerParams(
            needs_layout_passes=False, use_tc_tiling_on_sc=False),
        out_type=jax.ShapeDtypeStruct((N_EDGES,), jnp.float32),
        scratch_types=[
            pltpu.VMEM((HID, N_NODES), jnp.int32),
            pltpu.VMEM((per_w,), jnp.int32),
            pltpu.VMEM((per_w,), jnp.int32),
            pltpu.VMEM((per_w,), jnp.float32),
            pltpu.VMEM((160, 16), jnp.float32),
            pltpu.SemaphoreType.DMA,
            pltpu.SemaphoreType.DMA,
            pltpu.SemaphoreType.DMA,
            pltpu.SemaphoreType.DMA,
        ],
    )
    def sc_k(tab_h, src_h, dst_h, wpack_h, out_h,
             tabv, idx_s, idx_d, outb, wv, sem0, sem1, sem2, sem3):
        wid = lax.axis_index("s") * 2 + lax.axis_index("c")
        base = wid * per_w
        cps = [
            pltpu.async_copy(tab_h, tabv, sem0),
            pltpu.async_copy(src_h.at[pl.ds(base, per_w)], idx_s, sem1),
            pltpu.async_copy(dst_h.at[pl.ds(base, per_w)], idx_d, sem2),
            pltpu.async_copy(wpack_h, wv, sem3),
        ]
        for cp in cps:
            cp.wait()

        col = [jnp.full((16,), i, jnp.int32) for i in range(HID)]

        def edge_group_t(g):
            sv = idx_s[pl.ds(g * 16, 16)]
            dv = idx_d[pl.ds(g * 16, 16)]
            t = []
            for i in range(HID // 2):
                gs = plsc.load_gather(tabv, [col[i], sv])
                gd = plsc.load_gather(tabv, [col[HID // 2 + i], dv])
                s0, s1 = plsc.unpack(
                    plsc.bitcast(gs, jnp.bfloat16),
                    format=plsc.PackFormat.INTERLEAVED)
                d0, d1 = plsc.unpack(
                    plsc.bitcast(gd, jnp.bfloat16),
                    format=plsc.PackFormat.INTERLEAVED)
                e0 = jnp.exp(s0 + d0)
                t.append(1.0 / (e0 + 1.0))
                e1 = jnp.exp(s1 + d1)
                t.append(1.0 / (e1 + 1.0))
            return t

        def mlp_tail(ts):
            h = ts
            for wbase, bbase in ((0, 136), (64, 144)):
                nxt = [[] for _ in h]
                for j in range(HID):
                    bj = wv[bbase + j]
                    accs = [bj for _ in h]
                    for i in range(HID):
                        wij = wv[wbase + i * HID + j]
                        accs = [a + hg[i] * wij for a, hg in zip(accs, h)]
                    for k, a in enumerate(accs):
                        e = jnp.exp(a)
                        nxt[k].append(1.0 / (e + 1.0))
                h = nxt
            b4 = wv[152]
            outs = [b4 for _ in h]
            for i in range(HID):
                w4i = wv[128 + i]
                outs = [o + hg[i] * w4i for o, hg in zip(outs, h)]
            return outs

        pairs = groups // 2

        @plsc.parallel_loop(0, pairs, unroll=2)
        def pair_body(p):
            g0 = p * 2
            o0, o1 = mlp_tail([edge_group_t(g0), edge_group_t(g0 + 1)])
            outb[pl.ds(g0 * 16, 16)] = o0
            outb[pl.ds(g0 * 16 + 16, 16)] = o1

        if groups % 2:
            g = groups - 1
            (o_tail,) = mlp_tail([edge_group_t(g)])
            outb[pl.ds(g * 16, 16)] = o_tail

        pltpu.sync_copy(outb, out_h.at[pl.ds(base, per_w)])

    return sc_k(tabp, src, dst, wpack)


def kernel(inputs, edge_index, W1, b1, W2, b2, W3, b3, W4, b4):
    w1cat = jnp.concatenate([W1[:D_FEAT], W1[D_FEAT:]], axis=1)
    brow = jnp.concatenate([b1, jnp.zeros((HID,), jnp.float32)])[None, :]
    tab = _make_table(inputs, w1cat, brow)

    h16 = lax.bitcast_convert_type(
        tab.astype(jnp.bfloat16), jnp.uint16).astype(jnp.uint32)
    packed = h16[:, 0::2] | (h16[:, 1::2] << 16)
    tabp = lax.bitcast_convert_type(packed.T, jnp.int32)

    wflat = jnp.concatenate([
        -4.0 * W2.reshape(-1), -4.0 * W3.reshape(-1), -2.0 * W4.reshape(-1),
        2.0 * b2 + 2.0 * jnp.sum(W2, axis=0),
        2.0 * b3 + 2.0 * jnp.sum(W3, axis=0),
        b4 + jnp.sum(W4)[None], jnp.zeros((7,), jnp.float32),
    ])
    wpack = jnp.broadcast_to(wflat[:, None], (160, 16))

    per_w = N_EDGES // 32
    out = _sc_edge_mlp(tabp, edge_index[0], edge_index[1], wpack,
                       per_w=per_w)
    return out.reshape(N_EDGES, 1)
